# baseline (speedup 1.0000x reference)
import functools

import jax
import jax.numpy as jnp
from jax import lax
from jax.experimental import pallas as pl
from jax.experimental.pallas import tpu as pltpu
from jax.experimental.pallas import tpu_sc as plsc

D = 768
L = 16
NVR = D // L
NC, NS = 2, 16
NW = NC * NS
EPS = 1e-12
CHUNK = 64


def _rsqrt_v(x):
    i = plsc.bitcast(x, jnp.int32)
    i = jnp.int32(0x5F3759DF) - (i >> 1)
    y = plsc.bitcast(i, jnp.float32)
    for _ in range(3):
        y = y * (1.5 - 0.5 * x * y * y)
    return y


def _make_kernel(B):
    assert B % (NW * CHUNK) == 0
    b_per_w = B // NW
    n_chunks = b_per_w // CHUNK
    mesh = plsc.VectorSubcoreMesh(core_axis_name="c", subcore_axis_name="s")

    @functools.partial(
        pl.kernel,
        mesh=mesh,
        out_type=jax.ShapeDtypeStruct((B, D), jnp.float32),
        compiler_params=pltpu.CompilerParams(needs_layout_passes=False),
        scratch_types=[
            pltpu.VMEM((CHUNK,), jnp.int32),
            pltpu.VMEM((CHUNK,), jnp.int32),
            pltpu.VMEM((CHUNK, D), jnp.float32),
            pltpu.VMEM((CHUNK, D), jnp.float32),
            pltpu.VMEM((D,), jnp.float32),
            pltpu.VMEM((D,), jnp.float32),
            pltpu.SemaphoreType.DMA,
            pltpu.SemaphoreType.DMA,
        ],
    )
    def emb_ln(ids_hbm, table_hbm, lnw_hbm, lnb_hbm, out_hbm,
               idx0, idx1, rows0, rows1, lnw_v, lnb_v, sem0, sem1):
        wid = lax.axis_index("s") * NC + lax.axis_index("c")
        base = wid * b_per_w

        pltpu.sync_copy(lnw_hbm, lnw_v)
        pltpu.sync_copy(lnb_hbm, lnb_v)

        idx_bufs = (idx0, idx1)
        rows_bufs = (rows0, rows1)
        sems = (sem0, sem1)

        def start_gather(b, ch):
            off = pl.multiple_of(base + ch * CHUNK, CHUNK)
            pltpu.sync_copy(ids_hbm.at[pl.ds(off, CHUNK)], idx_bufs[b])
            pltpu.async_copy(table_hbm.at[idx_bufs[b]], rows_bufs[b], sems[b])

        start_gather(0, 0)
        start_gather(1, 1)

        def ln_chunk(rows):
          def ln_row(r, carry):
            acc = jnp.zeros((L,), jnp.float32)
            acc2 = jnp.zeros((L,), jnp.float32)
            for c in range(NVR):
                x = rows[r, pl.ds(c * L, L)]
                acc = acc + x
                acc2 = acc2 + x * x
            s = jnp.broadcast_to(jnp.sum(acc), (L,))
            s2 = jnp.broadcast_to(jnp.sum(acc2), (L,))
            mean = s * (1.0 / D)
            var = s2 * (1.0 / D) - mean * mean
            rstd = _rsqrt_v(var + EPS)
            for c in range(NVR):
                x = rows[r, pl.ds(c * L, L)]
                y = (x - mean) * rstd
                w = lnw_v[pl.ds(c * L, L)]
                bb = lnb_v[pl.ds(c * L, L)]
                rows[r, pl.ds(c * L, L)] = y * w + bb
            return carry
          lax.fori_loop(0, CHUNK, ln_row, 0, unroll=False)

        def step(j, carry):
            for b in range(2):
                ch = 2 * j + b
                rows = rows_bufs[b]
                pltpu.make_async_copy(
                    table_hbm.at[idx_bufs[b]], rows, sems[b]).wait()
                ln_chunk(rows)
                off = pl.multiple_of(base + ch * CHUNK, CHUNK)
                pltpu.sync_copy(rows, out_hbm.at[pl.ds(off, CHUNK)])

                @pl.when(j < n_chunks // 2 - 1)
                def _():
                    start_gather(b, ch + 2)
            return carry

        lax.fori_loop(0, n_chunks // 2, step, 0, unroll=False)

    return emb_ln


def kernel(input_ids, word_embeddings, ln_weight, ln_bias):
    shape = input_ids.shape
    B = shape[0] * shape[1]
    ids = input_ids.reshape(B).astype(jnp.int32)
    out = _make_kernel(B)(ids, word_embeddings, ln_weight, ln_bias)
    return out.reshape(shape + (D,))


# speedup vs baseline: 2.1629x; 2.1629x over previous
import functools

import jax
import jax.numpy as jnp
from jax import lax
from jax.experimental import pallas as pl
from jax.experimental.pallas import tpu as pltpu
from jax.experimental.pallas import tpu_sc as plsc

D = 768
L = 16
NVR = D // L
NC, NS = 2, 16
NW = NC * NS
EPS = 1e-12
CHUNK = 64


def _rsqrt_v(x):
    i = plsc.bitcast(x, jnp.int32)
    i = jnp.int32(0x5F3759DF) - (i >> 1)
    y = plsc.bitcast(i, jnp.float32)
    for _ in range(2):
        y = y * (1.5 - 0.5 * x * y * y)
    return y


def _make_kernel(B):
    assert B % (NW * CHUNK) == 0
    b_per_w = B // NW
    n_chunks = b_per_w // CHUNK
    mesh = plsc.VectorSubcoreMesh(core_axis_name="c", subcore_axis_name="s")

    @functools.partial(
        pl.kernel,
        mesh=mesh,
        out_type=jax.ShapeDtypeStruct((B, D), jnp.float32),
        compiler_params=pltpu.CompilerParams(needs_layout_passes=False),
        scratch_types=[
            pltpu.VMEM((CHUNK,), jnp.int32),
            pltpu.VMEM((CHUNK,), jnp.int32),
            pltpu.VMEM((CHUNK, D), jnp.float32),
            pltpu.VMEM((CHUNK, D), jnp.float32),
            pltpu.VMEM((D,), jnp.float32),
            pltpu.VMEM((D,), jnp.float32),
            pltpu.SemaphoreType.DMA,
            pltpu.SemaphoreType.DMA,
        ],
    )
    def emb_ln(ids_hbm, table_hbm, lnw_hbm, lnb_hbm, out_hbm,
               idx0, idx1, rows0, rows1, lnw_v, lnb_v, sem0, sem1):
        wid = lax.axis_index("s") * NC + lax.axis_index("c")
        base = wid * b_per_w

        pltpu.sync_copy(lnw_hbm, lnw_v)
        pltpu.sync_copy(lnb_hbm, lnb_v)

        idx_bufs = (idx0, idx1)
        rows_bufs = (rows0, rows1)
        sems = (sem0, sem1)

        def start_gather(b, ch):
            off = pl.multiple_of(base + ch * CHUNK, CHUNK)
            pltpu.sync_copy(ids_hbm.at[pl.ds(off, CHUNK)], idx_bufs[b])
            pltpu.async_copy(table_hbm.at[idx_bufs[b]], rows_bufs[b], sems[b])

        start_gather(0, 0)
        start_gather(1, 1)

        ROWU = 4

        def ln_chunk(rows):
          def ln_quad(q, carry):
            r0 = q * ROWU
            acc = [jnp.zeros((L,), jnp.float32) for _ in range(ROWU)]
            acc2 = [jnp.zeros((L,), jnp.float32) for _ in range(ROWU)]
            for c in range(NVR):
                for r in range(ROWU):
                    x = rows[r0 + r, pl.ds(c * L, L)]
                    acc[r] = acc[r] + x
                    acc2[r] = acc2[r] + x * x
            mean = []
            rstd = []
            for r in range(ROWU):
                s = jnp.broadcast_to(jnp.sum(acc[r]), (L,))
                s2 = jnp.broadcast_to(jnp.sum(acc2[r]), (L,))
                m = s * (1.0 / D)
                var = s2 * (1.0 / D) - m * m
                mean.append(m)
                rstd.append(_rsqrt_v(var + EPS))
            for c in range(NVR):
                w = lnw_v[pl.ds(c * L, L)]
                bb = lnb_v[pl.ds(c * L, L)]
                for r in range(ROWU):
                    x = rows[r0 + r, pl.ds(c * L, L)]
                    y = (x - mean[r]) * rstd[r]
                    rows[r0 + r, pl.ds(c * L, L)] = y * w + bb
            return carry
          lax.fori_loop(0, CHUNK // ROWU, ln_quad, 0, unroll=False)

        def step(j, carry):
            for b in range(2):
                ch = 2 * j + b
                rows = rows_bufs[b]
                pltpu.make_async_copy(
                    table_hbm.at[idx_bufs[b]], rows, sems[b]).wait()
                ln_chunk(rows)
                off = pl.multiple_of(base + ch * CHUNK, CHUNK)
                pltpu.sync_copy(rows, out_hbm.at[pl.ds(off, CHUNK)])

                @pl.when(j < n_chunks // 2 - 1)
                def _():
                    start_gather(b, ch + 2)
            return carry

        lax.fori_loop(0, n_chunks // 2, step, 0, unroll=False)

    return emb_ln


def kernel(input_ids, word_embeddings, ln_weight, ln_bias):
    shape = input_ids.shape
    B = shape[0] * shape[1]
    ids = input_ids.reshape(B).astype(jnp.int32)
    out = _make_kernel(B)(ids, word_embeddings, ln_weight, ln_bias)
    return out.reshape(shape + (D,))


# speedup vs baseline: 2.3620x; 1.0921x over previous
import functools

import jax
import jax.numpy as jnp
from jax import lax
from jax.experimental import pallas as pl
from jax.experimental.pallas import tpu as pltpu
from jax.experimental.pallas import tpu_sc as plsc

D = 768
L = 16
NVR = D // L
NC, NS = 2, 16
NW = NC * NS
EPS = 1e-12
CHUNK = 32
NBUF = 4


def _rsqrt_v(x):
    i = plsc.bitcast(x, jnp.int32)
    i = jnp.int32(0x5F3759DF) - (i >> 1)
    y = plsc.bitcast(i, jnp.float32)
    for _ in range(2):
        y = y * (1.5 - 0.5 * x * y * y)
    return y


def _make_kernel(B):
    assert B % (NW * CHUNK) == 0
    b_per_w = B // NW
    n_chunks = b_per_w // CHUNK
    mesh = plsc.VectorSubcoreMesh(core_axis_name="c", subcore_axis_name="s")

    @functools.partial(
        pl.kernel,
        mesh=mesh,
        out_type=jax.ShapeDtypeStruct((B, D), jnp.float32),
        compiler_params=pltpu.CompilerParams(needs_layout_passes=False),
        scratch_types=(
            [pltpu.VMEM((CHUNK,), jnp.int32)] * NBUF
            + [pltpu.VMEM((CHUNK, D), jnp.float32)] * NBUF
            + [pltpu.VMEM((D,), jnp.float32)] * 2
            + [pltpu.SemaphoreType.DMA] * (2 * NBUF)
        ),
    )
    def emb_ln(ids_hbm, table_hbm, lnw_hbm, lnb_hbm, out_hbm, *scratch):
        idx_bufs = scratch[:NBUF]
        rows_bufs = scratch[NBUF:2 * NBUF]
        lnw_v, lnb_v = scratch[2 * NBUF:2 * NBUF + 2]
        gsems = scratch[2 * NBUF + 2:3 * NBUF + 2]
        wsems = scratch[3 * NBUF + 2:]
        wid = lax.axis_index("s") * NC + lax.axis_index("c")
        base = wid * b_per_w

        pltpu.sync_copy(lnw_hbm, lnw_v)
        pltpu.sync_copy(lnb_hbm, lnb_v)

        def start_gather(b, ch):
            off = pl.multiple_of(base + ch * CHUNK, CHUNK)
            pltpu.sync_copy(ids_hbm.at[pl.ds(off, CHUNK)], idx_bufs[b])
            pltpu.async_copy(table_hbm.at[idx_bufs[b]], rows_bufs[b], gsems[b])

        def wait_gather(b):
            pltpu.make_async_copy(
                table_hbm.at[idx_bufs[b]], rows_bufs[b], gsems[b]).wait()

        def start_write(b, ch):
            off = pl.multiple_of(base + ch * CHUNK, CHUNK)
            pltpu.async_copy(
                rows_bufs[b], out_hbm.at[pl.ds(off, CHUNK)], wsems[b])

        def wait_write(b, ch):
            off = pl.multiple_of(base + ch * CHUNK, CHUNK)
            pltpu.make_async_copy(
                rows_bufs[b], out_hbm.at[pl.ds(off, CHUNK)], wsems[b]).wait()

        for b in range(NBUF - 1):
            start_gather(b, b)

        ROWU = 4

        def ln_chunk(rows):
          def ln_quad(q, carry):
            r0 = q * ROWU
            acc = [jnp.zeros((L,), jnp.float32) for _ in range(ROWU)]
            acc2 = [jnp.zeros((L,), jnp.float32) for _ in range(ROWU)]
            for c in range(NVR):
                for r in range(ROWU):
                    x = rows[r0 + r, pl.ds(c * L, L)]
                    acc[r] = acc[r] + x
                    acc2[r] = acc2[r] + x * x
            mean = []
            rstd = []
            for r in range(ROWU):
                s = jnp.broadcast_to(jnp.sum(acc[r]), (L,))
                s2 = jnp.broadcast_to(jnp.sum(acc2[r]), (L,))
                m = s * (1.0 / D)
                var = s2 * (1.0 / D) - m * m
                mean.append(m)
                rstd.append(_rsqrt_v(var + EPS))
            for c in range(NVR):
                w = lnw_v[pl.ds(c * L, L)]
                bb = lnb_v[pl.ds(c * L, L)]
                for r in range(ROWU):
                    x = rows[r0 + r, pl.ds(c * L, L)]
                    y = (x - mean[r]) * rstd[r]
                    rows[r0 + r, pl.ds(c * L, L)] = y * w + bb
            return carry
          lax.fori_loop(0, CHUNK // ROWU, ln_quad, 0, unroll=False)

        n_outer = n_chunks // NBUF

        def step(j, carry):
            for b in range(NBUF):
                ch = j * NBUF + b
                wait_gather(b)
                ln_chunk(rows_bufs[b])
                start_write(b, ch)
                pb = (b + NBUF - 1) % NBUF

                def reclaim():
                    wait_write(pb, ch - 1)
                    start_gather(pb, ch + NBUF - 1)

                if b == 0:
                    @pl.when(j > 0)
                    def _():
                        wait_write(pb, ch - 1)

                    start_gather(pb, ch + NBUF - 1)
                else:
                    @pl.when(j < n_outer - 1)
                    def _():
                        reclaim()

                    @pl.when(j == n_outer - 1)
                    def _():
                        wait_write(pb, ch - 1)
            return carry

        lax.fori_loop(0, n_outer, step, 0, unroll=False)
        wait_write(NBUF - 1, n_chunks - 1)

    return emb_ln


def kernel(input_ids, word_embeddings, ln_weight, ln_bias):
    shape = input_ids.shape
    B = shape[0] * shape[1]
    ids = input_ids.reshape(B).astype(jnp.int32)
    out = _make_kernel(B)(ids, word_embeddings, ln_weight, ln_bias)
    return out.reshape(shape + (D,))
